# baseline (speedup 1.0000x reference)
import jax
import jax.numpy as jnp
from jax import lax
from jax.experimental import pallas as pl
from jax.experimental.pallas import tpu as pltpu
from jax.experimental.pallas import tpu_sc as plsc

SEQ_LEN = 2048
BATCH = 8
NEG = -1000000000.0

ROWS_PER_BLOCK = 8
NUM_BLOCKS = (BATCH * SEQ_LEN) // ROWS_PER_BLOCK
NUM_WORKERS = 32
BLOCKS_PER_WORKER = NUM_BLOCKS // NUM_WORKERS
ROW_BLOCKS_PER_BATCH = SEQ_LEN // ROWS_PER_BLOCK
NBUF = 2
CHUNKS = (2048, 256, 128)
STRIP = 128
NEGW = CHUNKS[0]


def _chunk_ops(do_wait, lo, hi, make_desc):
    off0 = lo
    for size in CHUNKS:
        span = hi - off0
        k = span // size

        def body(i, off):
            d = make_desc(pl.multiple_of(off, CHUNKS[-1]), size)
            if do_wait:
                d.wait()
            else:
                d.start()
            return off + size

        off0 = lax.fori_loop(0, k, body, off0)


def _sc_body(x_hbm, n_hbm, out_hbm, n_stage, n_sm, bufs, negb, sem_n,
             sems_rd, sems_wr):
    cid = lax.axis_index("c")
    sid = lax.axis_index("s")
    wid = sid * 2 + cid

    pltpu.async_copy(n_hbm, n_stage.at[pl.ds(0, BATCH)], sem_n).wait()
    nv = n_stage[pl.ds(0, 16)]
    for i in range(BATCH):
        n_sm[i] = nv[i]
    negv = jnp.full((16,), NEG, dtype=jnp.float32)
    for r in range(ROWS_PER_BLOCK):
        for c in range(NEGW // 16):
            negb[r, pl.ds(c * 16, 16)] = negv

    iota = lax.iota(jnp.int32, 16)

    def n_at(b):
        return n_sm[b]

    def process(g, slot):
        blk = wid + NUM_WORKERS * g
        b = blk // ROW_BLOCKS_PER_BATCH
        r0 = (blk % ROW_BLOCKS_PER_BATCH) * ROWS_PER_BLOCK
        nb = n_at(b)
        buf = bufs[slot]
        sem_wr = sems_wr[slot]
        sem_rd = sems_rd[slot]

        @pl.when(g >= NBUF)
        def _():
            pltpu.make_async_copy(
                x_hbm.at[0, pl.ds(0, ROWS_PER_BLOCK), :], buf, sem_wr
            ).wait()

        w = jnp.where(r0 >= nb, 0, (nb // STRIP) * STRIP + STRIP)

        def neg_desc(base):
            def make(off, size):
                return pltpu.make_async_copy(
                    negb.at[:, pl.ds(0, size)],
                    out_hbm.at[
                        b,
                        pl.ds(r0, ROWS_PER_BLOCK),
                        pl.ds(pl.multiple_of(base + off, CHUNKS[-1]), size),
                    ],
                    sem_wr,
                )
            return make

        def rd_desc(off, size):
            return pltpu.make_async_copy(
                x_hbm.at[b, pl.ds(r0, ROWS_PER_BLOCK), pl.ds(off, size)],
                buf.at[:, pl.ds(off, size)],
                sem_rd,
            )

        @pl.when(w > 0)
        def _():
            _chunk_ops(False, 0, w, rd_desc)

        _chunk_ops(False, 0, SEQ_LEN - w, neg_desc(w))

        @pl.when(w > 0)
        def _():
            _chunk_ops(True, 0, w, rd_desc)

            straddle = nb < r0 + ROWS_PER_BLOCK
            lo = jnp.where(straddle, 0, w - STRIP)
            cnt = (w - lo) // 16
            for r in range(ROWS_PER_BLOCK):
                lim = jnp.where((r0 + r) < nb, nb, 0)
                cb = jnp.clip((lim - lo) // 16, 0, cnt)

                @pl.when(cb < cnt)
                def _():
                    off = pl.multiple_of(lo + cb * 16, 16)
                    v = buf[r, pl.ds(off, 16)]
                    m = (iota + off) < lim
                    buf[r, pl.ds(off, 16)] = jnp.where(m, v, negv)

                def mbody(c, off):
                    buf[r, pl.ds(pl.multiple_of(off, 16), 16)] = negv
                    return off + 16

                lax.fori_loop(cb + 1, cnt, mbody, lo + (cb + 1) * 16)

            def wr_desc(off, size):
                return pltpu.make_async_copy(
                    buf.at[:, pl.ds(off, size)],
                    out_hbm.at[b, pl.ds(r0, ROWS_PER_BLOCK), pl.ds(off, size)],
                    sem_wr,
                )

            _chunk_ops(False, 0, w, wr_desc)

    def outer(g2, _):
        for j in range(NBUF):
            process(g2 * NBUF + j, j)
        return 0

    lax.fori_loop(0, BLOCKS_PER_WORKER // NBUF, outer, 0)

    for j in range(NBUF):
        pltpu.make_async_copy(
            x_hbm.at[0, pl.ds(0, ROWS_PER_BLOCK), :], bufs[j], sems_wr[j]
        ).wait()


def kernel(x, n):
    mesh = plsc.VectorSubcoreMesh(
        core_axis_name="c", subcore_axis_name="s", num_cores=2, num_subcores=16
    )

    def body(x_hbm, n_hbm, out_hbm, n_stage, n_sm, buf0, buf1, negb, sem_n,
             sem_rd0, sem_rd1, sem_wr0, sem_wr1):
        _sc_body(x_hbm, n_hbm, out_hbm, n_stage, n_sm, (buf0, buf1), negb,
                 sem_n, (sem_rd0, sem_rd1), (sem_wr0, sem_wr1))

    f = pl.kernel(
        body,
        out_type=jax.ShapeDtypeStruct((BATCH, SEQ_LEN, SEQ_LEN), jnp.float32),
        mesh=mesh,
        scratch_types=[
            pltpu.VMEM((16,), jnp.int32),
            pltpu.SMEM((BATCH,), jnp.int32),
            pltpu.VMEM((ROWS_PER_BLOCK, SEQ_LEN), jnp.float32),
            pltpu.VMEM((ROWS_PER_BLOCK, SEQ_LEN), jnp.float32),
            pltpu.VMEM((ROWS_PER_BLOCK, NEGW), jnp.float32),
            pltpu.SemaphoreType.DMA,
            pltpu.SemaphoreType.DMA,
            pltpu.SemaphoreType.DMA,
            pltpu.SemaphoreType.DMA,
            pltpu.SemaphoreType.DMA,
        ],
    )
    return f(x, n.astype(jnp.int32))


# speedup vs baseline: 1.1295x; 1.1295x over previous
import jax
import jax.numpy as jnp
from jax import lax
from jax.experimental import pallas as pl
from jax.experimental.pallas import tpu as pltpu
from jax.experimental.pallas import tpu_sc as plsc

SEQ_LEN = 2048
BATCH = 8
NEG = -1000000000.0

ROWS_PER_BLOCK = 16
NUM_BLOCKS = (BATCH * SEQ_LEN) // ROWS_PER_BLOCK
NUM_WORKERS = 32
BLOCKS_PER_WORKER = NUM_BLOCKS // NUM_WORKERS
ROW_BLOCKS_PER_BATCH = SEQ_LEN // ROWS_PER_BLOCK
NBUF = 3
CHUNKS = (256, 128)
STRIP = 128
NEGW = CHUNKS[0]
MAIN_BLOCKS = (BLOCKS_PER_WORKER // NBUF) * NBUF


def _chunk_ops(do_wait, lo, hi, make_desc):
    off0 = lo
    for size in CHUNKS:
        span = hi - off0
        k = span // size

        def body(i, off):
            d = make_desc(pl.multiple_of(off, CHUNKS[-1]), size)
            if do_wait:
                d.wait()
            else:
                d.start()
            return off + size

        off0 = lax.fori_loop(0, k, body, off0)


def _sc_body(x_hbm, n_hbm, out_hbm, n_stage, n_sm, bufs, negb, sem_n,
             sems_rd, sems_wr):
    cid = lax.axis_index("c")
    sid = lax.axis_index("s")
    wid = sid * 2 + cid

    pltpu.async_copy(n_hbm, n_stage.at[pl.ds(0, BATCH)], sem_n).wait()
    nv = n_stage[pl.ds(0, 16)]
    for i in range(BATCH):
        n_sm[i] = nv[i]
    negv = jnp.full((16,), NEG, dtype=jnp.float32)
    for r in range(ROWS_PER_BLOCK):
        for c in range(NEGW // 16):
            negb[r, pl.ds(c * 16, 16)] = negv

    iota = lax.iota(jnp.int32, 16)

    def n_at(b):
        return n_sm[b]

    def geom(g):
        blk = wid + NUM_WORKERS * g
        b = blk // ROW_BLOCKS_PER_BATCH
        r0 = (blk % ROW_BLOCKS_PER_BATCH) * ROWS_PER_BLOCK
        nb = n_at(b)
        w = jnp.where(r0 >= nb, 0, (nb // STRIP) * STRIP + STRIP)
        return b, r0, nb, w

    def mk_rd_desc(g, slot):
        b, r0, nb, w = geom(g)

        def rd(off, size):
            return pltpu.make_async_copy(
                x_hbm.at[b, pl.ds(r0, ROWS_PER_BLOCK), pl.ds(off, size)],
                bufs[slot].at[:, pl.ds(off, size)],
                sems_rd[slot],
            )
        return w, rd

    def drain_writes(slot):
        pltpu.make_async_copy(
            x_hbm.at[0, pl.ds(0, ROWS_PER_BLOCK), :], bufs[slot],
            sems_wr[slot],
        ).wait()

    def issue_reads(g, slot):
        w, rd = mk_rd_desc(g, slot)
        _chunk_ops(False, 0, w, rd)

    def process(g, slot):
        b, r0, nb, w = geom(g)
        buf = bufs[slot]
        sem_wr = sems_wr[slot]

        def neg_desc(base):
            def make(off, size):
                return pltpu.make_async_copy(
                    negb.at[:, pl.ds(0, size)],
                    out_hbm.at[
                        b,
                        pl.ds(r0, ROWS_PER_BLOCK),
                        pl.ds(pl.multiple_of(base + off, CHUNKS[-1]), size),
                    ],
                    sem_wr,
                )
            return make

        _chunk_ops(False, 0, SEQ_LEN - w, neg_desc(w))

        @pl.when(w > 0)
        def _():
            _, rd = mk_rd_desc(g, slot)
            _chunk_ops(True, 0, w, rd)

            straddle = nb < r0 + ROWS_PER_BLOCK
            lo = jnp.where(straddle, 0, w - STRIP)
            cnt = (w - lo) // 16
            for r in range(ROWS_PER_BLOCK):
                lim = jnp.where((r0 + r) < nb, nb, 0)
                cb = jnp.clip((lim - lo) // 16, 0, cnt)

                @pl.when(cb < cnt)
                def _():
                    off = pl.multiple_of(lo + cb * 16, 16)
                    v = buf[r, pl.ds(off, 16)]
                    m = (iota + off) < lim
                    buf[r, pl.ds(off, 16)] = jnp.where(m, v, negv)

                def mbody(c, off):
                    buf[r, pl.ds(pl.multiple_of(off, 16), 16)] = negv
                    return off + 16

                lax.fori_loop(cb + 1, cnt, mbody, lo + (cb + 1) * 16)

            def wr_desc(off, size):
                return pltpu.make_async_copy(
                    buf.at[:, pl.ds(off, size)],
                    out_hbm.at[b, pl.ds(r0, ROWS_PER_BLOCK), pl.ds(off, size)],
                    sem_wr,
                )

            _chunk_ops(False, 0, w, wr_desc)

    def step(g, j):
        la = g + NBUF - 1
        la_slot = (j + NBUF - 1) % NBUF

        @pl.when(la >= NBUF)
        def _():
            drain_writes(la_slot)

        @pl.when(la < BLOCKS_PER_WORKER)
        def _():
            issue_reads(la, la_slot)

        process(g, j)

    for j in range(NBUF - 1):
        issue_reads(j, j)

    def outer(g2, _):
        for j in range(NBUF):
            step(g2 * NBUF + j, j)
        return 0

    lax.fori_loop(0, MAIN_BLOCKS // NBUF, outer, 0)

    for g in range(MAIN_BLOCKS, BLOCKS_PER_WORKER):
        step(g, g % NBUF)
    drain_writes((BLOCKS_PER_WORKER - 1) % NBUF)


def kernel(x, n):
    mesh = plsc.VectorSubcoreMesh(
        core_axis_name="c", subcore_axis_name="s", num_cores=2, num_subcores=16
    )

    def body(x_hbm, n_hbm, out_hbm, n_stage, n_sm, buf0, buf1, buf2, negb,
             sem_n, sem_rd0, sem_rd1, sem_rd2, sem_wr0, sem_wr1, sem_wr2):
        _sc_body(x_hbm, n_hbm, out_hbm, n_stage, n_sm, (buf0, buf1, buf2),
                 negb, sem_n, (sem_rd0, sem_rd1, sem_rd2),
                 (sem_wr0, sem_wr1, sem_wr2))

    f = pl.kernel(
        body,
        out_type=jax.ShapeDtypeStruct((BATCH, SEQ_LEN, SEQ_LEN), jnp.float32),
        mesh=mesh,
        scratch_types=[
            pltpu.VMEM((16,), jnp.int32),
            pltpu.SMEM((BATCH,), jnp.int32),
            pltpu.VMEM((ROWS_PER_BLOCK, SEQ_LEN), jnp.float32),
            pltpu.VMEM((ROWS_PER_BLOCK, SEQ_LEN), jnp.float32),
            pltpu.VMEM((ROWS_PER_BLOCK, SEQ_LEN), jnp.float32),
            pltpu.VMEM((ROWS_PER_BLOCK, NEGW), jnp.float32),
            pltpu.SemaphoreType.DMA,
            pltpu.SemaphoreType.DMA,
            pltpu.SemaphoreType.DMA,
            pltpu.SemaphoreType.DMA,
            pltpu.SemaphoreType.DMA,
            pltpu.SemaphoreType.DMA,
            pltpu.SemaphoreType.DMA,
        ],
    )
    return f(x, n.astype(jnp.int32))
